# baseline (speedup 1.0000x reference)
import functools

import jax
import jax.numpy as jnp
import numpy as np
from jax import lax
from jax.experimental import pallas as pl
from jax.experimental.pallas import tpu as pltpu
from jax.experimental.pallas import tpu_sc as plsc

_B, _H, _W, _C = 4, 112, 112, 96
_HO, _WO = 224, 224
_N = _H * _W * _C
_F = _HO * _WO * _C
_E = _B * _N
_NW = 32
_PER_W = _E // _NW
_KS = 1344
_NBUF = 8
_NGRP = _PER_W // (_KS * _NBUF)

_RECIP224 = np.float32((1.0 + 2.0**-21) / 224.0)
_RECIP96 = np.float32(1.0 / 96.0)

_mesh = plsc.VectorSubcoreMesh(core_axis_name="c", subcore_axis_name="s")


@functools.partial(
    pl.kernel,
    out_type=(),
    mesh=_mesh,
    scratch_types=[pltpu.VMEM((_KS,), jnp.int32)] * _NBUF
    + [pltpu.VMEM((_KS,), jnp.int32)] * _NBUF
    + [pltpu.VMEM((_KS,), jnp.int32)] * _NBUF
    + [pltpu.VMEM((_KS,), jnp.float32)] * _NBUF
    + [pltpu.SemaphoreType.DMA] * _NBUF,
)
def _unpool_scatter(pooled_hbm, idx_hbm, out_ref, *scratch):
    idx_v = scratch[0:_NBUF]
    src_v = scratch[_NBUF:2 * _NBUF]
    dst_v = scratch[2 * _NBUF:3 * _NBUF]
    val_v = scratch[3 * _NBUF:4 * _NBUF]
    sems = scratch[4 * _NBUF:5 * _NBUF]
    wid = lax.axis_index("s") * 2 + lax.axis_index("c")
    b = wid >> 3
    base = wid * _PER_W
    src_off = b * _N
    dst_off = b * _F

    def group_body(g, carry):
        gbase = base + g * (_KS * _NBUF)
        idx_copies = [
            pltpu.async_copy(
                idx_hbm.at[pl.ds(gbase + s * _KS, _KS)], idx_v[s], sems[s]
            )
            for s in range(_NBUF)
        ]
        gathers = []
        for s in range(_NBUF):
            idx_copies[s].wait()
            idx_row, src_row, dst_row = idx_v[s], src_v[s], dst_v[s]

            def vec_body(j, carry2, idx_row=idx_row, src_row=src_row,
                         dst_row=dst_row):
                sl = pl.ds(pl.multiple_of(j * 16, 16), 16)
                i = idx_row[sl]
                fi = i.astype(jnp.float32)
                q = (fi * _RECIP96).astype(jnp.int32)
                c = i - q * 96
                qf = q.astype(jnp.float32)
                ho = (qf * _RECIP224).astype(jnp.int32)
                wo = q - ho * 224
                src = ((ho >> 1) * (_W * _C) + (wo >> 1) * _C + c) + src_off
                src_row[sl] = src
                dst_row[sl] = i + dst_off
                return carry2

            lax.fori_loop(0, _KS // 16, vec_body, 0, unroll=4)
            gathers.append(
                pltpu.async_copy(pooled_hbm.at[src_v[s]], val_v[s], sems[s])
            )
        scatters = []
        for s in range(_NBUF):
            gathers[s].wait()
            scatters.append(
                pltpu.async_copy(val_v[s], out_ref.at[dst_v[s]], sems[s])
            )
        for s in range(_NBUF):
            scatters[s].wait()
        return carry

    lax.fori_loop(0, _NGRP, group_body, 0)


def kernel(pooled_Maps, indices, Rectified_FM):
    del Rectified_FM
    pooled_flat = pooled_Maps.reshape(-1)
    idx_flat = indices.reshape(-1)
    out_ref = jax.new_ref(jnp.zeros((_B * _F,), jnp.float32))
    _unpool_scatter(pooled_flat, idx_flat, out_ref)
    return out_ref[...].reshape(_B, _HO, _WO, _C)
